# baseline (speedup 1.0000x reference)
import functools

import jax
import jax.numpy as jnp
from jax import lax
from jax.experimental import pallas as pl
from jax.experimental.pallas import tpu as pltpu
from jax.experimental.pallas import tpu_sc as plsc

NC = 2
NS = 16
L = 16
NW = NC * NS
CHUNK = 128
DP = 24
DH = 20


def _mesh():
    return plsc.VectorSubcoreMesh(
        core_axis_name="c", subcore_axis_name="s", num_cores=NC, num_subcores=NS
    )


@functools.cache
def _deg_kernel(n_pad, pt):
    def body(src_hbm, dst_hbm, ew_hbm, dp_hbm, src_v, dst_v, ew_v,
             acc0, acc1, acc2):
        cid = lax.axis_index("c")
        sid = lax.axis_index("s")
        wid = sid * NC + cid
        z16 = jnp.zeros((L,), jnp.float32)

        def zero_body(i, carry):
            acc0[pl.ds(i * L, L)] = z16
            acc1[pl.ds(i * L, L)] = z16
            acc2[pl.ds(i * L, L)] = z16
            return carry

        lax.fori_loop(0, n_pad // L, zero_body, 0)

        pltpu.sync_copy(src_hbm.at[wid], src_v)
        pltpu.sync_copy(dst_hbm.at[wid], dst_v)
        pltpu.sync_copy(ew_hbm.at[wid], ew_v)

        ones = jnp.ones((L,), jnp.float32)

        def hist_body(i, carry):
            s16 = src_v[pl.ds(i * L, L)]
            d16 = dst_v[pl.ds(i * L, L)]
            w16 = ew_v[pl.ds(i * L, L)]
            plsc.addupdate_scatter(acc0, [s16], ones)
            plsc.addupdate_scatter(acc1, [d16], ones)
            plsc.addupdate_scatter(acc2, [d16], w16)
            return carry

        lax.fori_loop(0, pt // L, hist_body, 0)
        pltpu.sync_copy(acc0, dp_hbm.at[wid, 0])
        pltpu.sync_copy(acc1, dp_hbm.at[wid, 1])
        pltpu.sync_copy(acc2, dp_hbm.at[wid, 2])

    return pl.kernel(
        body,
        out_type=jax.ShapeDtypeStruct((NW, 3, n_pad), jnp.float32),
        mesh=_mesh(),
        compiler_params=pltpu.CompilerParams(needs_layout_passes=False, use_tc_tiling_on_sc=False),
        scratch_types=[
            pltpu.VMEM((pt,), jnp.int32),
            pltpu.VMEM((pt,), jnp.int32),
            pltpu.VMEM((pt,), jnp.float32),
            pltpu.VMEM((n_pad,), jnp.float32),
            pltpu.VMEM((n_pad,), jnp.float32),
            pltpu.VMEM((n_pad,), jnp.float32),
        ],
    )


@functools.cache
def _prop_kernel(n_pad, nch, with_ew):
    rpt = n_pad // NS
    assert nch % 4 == 0 and nch >= 8

    def body(x_hbm, src_hbm, dst_hbm, ew_hbm, zz_hbm, parts_hbm,
             sidx, didx, ewv, *scr):
        rows = scr[0:4]
        rows_m = scr[4:8] if with_ew else scr[0:4]
        acc = scr[-9]
        sg = scr[-8:-4]
        ss = scr[-4:]
        cid = lax.axis_index("c")
        sid = lax.axis_index("s")
        wid = sid * NC + cid

        pltpu.sync_copy(zz_hbm.at[pl.ds(sid * rpt, rpt)],
                        acc.at[pl.ds(sid * rpt, rpt)])
        pltpu.sync_copy(src_hbm.at[wid], sidx)
        pltpu.sync_copy(dst_hbm.at[wid], didx)
        if with_ew:
            pltpu.sync_copy(ew_hbm.at[wid], ewv)
        plsc.subcore_barrier()

        riota = lax.iota(jnp.int32, L)

        def gather(j, b):
            pltpu.async_copy(x_hbm.at[sidx.at[j]], rows[b], sg[b])

        def wait_gather(b):
            pltpu.make_async_copy(x_hbm.at[sidx.at[0]], rows[b], sg[b]).wait()

        def scatter(j, b):
            pltpu.async_copy(rows_m[b], acc.at[didx.at[j]], ss[b], add=True)

        def wait_scatter(b):
            pltpu.make_async_copy(rows_m[b], acc.at[didx.at[0]], ss[b]).wait()

        def mul_ew(j, b):
            for g in range(CHUNK // L):
                ew16 = ewv[j, pl.ds(g * L, L)]
                r16 = riota + jnp.int32(g * L)
                for c in range(DH):
                    c16 = jnp.full((L,), c, jnp.int32)
                    v = plsc.load_gather(rows[b], [r16, c16])
                    plsc.store_scatter(rows_m[b], [r16, c16], v * ew16)

        gather(0, 0)
        gather(1, 1)

        def outer(j0, carry):
            for db in range(4):
                j = j0 * 4 + db
                b = db
                wait_gather(b)
                if with_ew:
                    mul_ew(j, b)
                scatter(j, b)
                bn = (db + 2) % 4
                jn = j + 2

                def issue_next():
                    gather(jn, bn)

                if db >= 2:
                    @pl.when(jn < nch)
                    def _():
                        wait_scatter(bn)
                        issue_next()
                else:
                    @pl.when(jn < nch)
                    def _():
                        @pl.when(j0 >= 1)
                        def _():
                            wait_scatter(bn)
                        issue_next()
            return carry

        lax.fori_loop(0, nch // 4, outer, 0)
        for b in range(4):
            wait_scatter(b)
        plsc.subcore_barrier()
        pltpu.sync_copy(acc.at[pl.ds(sid * rpt, rpt)],
                        parts_hbm.at[cid, pl.ds(sid * rpt, rpt)])

    return pl.kernel(
        body,
        out_type=jax.ShapeDtypeStruct((NC, n_pad, DP), jnp.float32),
        mesh=_mesh(),
        compiler_params=pltpu.CompilerParams(needs_layout_passes=False, use_tc_tiling_on_sc=False),
        scratch_types=[
            pltpu.VMEM((nch, CHUNK), jnp.int32),
            pltpu.VMEM((nch, CHUNK), jnp.int32),
            pltpu.VMEM((nch, CHUNK), jnp.float32),
            *([pltpu.VMEM((CHUNK, DP), jnp.float32)] * (8 if with_ew else 4)),
            pltpu.VMEM_SHARED((n_pad, DP), jnp.float32),
            pltpu.SemaphoreType.DMA,
            pltpu.SemaphoreType.DMA,
            pltpu.SemaphoreType.DMA,
            pltpu.SemaphoreType.DMA,
            pltpu.SemaphoreType.DMA,
            pltpu.SemaphoreType.DMA,
            pltpu.SemaphoreType.DMA,
            pltpu.SemaphoreType.DMA,
        ],
    )


@functools.cache
def _combine_matmul1(n_pad, n, d_in):
    def body(dp_ref, feat_ref, w1_ref, x1_ref, scal_ref):
        d = jnp.sum(dp_ref[...], axis=0)
        s_out = lax.rsqrt(jnp.maximum(d[0], 1.0))
        s_in = lax.rsqrt(jnp.maximum(d[1], 1.0))
        x1_ref[pl.ds(0, n)] = jnp.dot(
            feat_ref[...] * s_out[:n, None], w1_ref[...],
            preferred_element_type=jnp.float32)
        x1_ref[pl.ds(n, n_pad - n)] = jnp.zeros((n_pad - n, DP), jnp.float32)
        scal_ref[...] = jnp.stack(
            [s_out, s_in, d[2], jnp.zeros_like(s_out)], axis=1)

    return pl.pallas_call(
        body,
        out_shape=(
            jax.ShapeDtypeStruct((n_pad, DP), jnp.float32),
            jax.ShapeDtypeStruct((n_pad, 4), jnp.float32),
        ),
    )


@functools.cache
def _mid_elementwise(n_pad, n):
    def body(parts_ref, scal_ref, b1_ref, x2_ref):
        p = parts_ref[0] + parts_ref[1]
        h = jnp.maximum(p * scal_ref[:, 1:2] + b1_ref[...][None, :], 0.0)
        h = h * scal_ref[:, 0:1]
        rmask = lax.broadcasted_iota(jnp.int32, (n_pad, 1), 0) < n
        x2_ref[...] = jnp.where(rmask, h, 0.0)

    return pl.pallas_call(
        body, out_shape=jax.ShapeDtypeStruct((n_pad, DP), jnp.float32))


@functools.cache
def _pre_final_scale(n_pad):
    def body(parts_ref, scal_ref, t_ref):
        t_ref[...] = (parts_ref[0] + parts_ref[1]) * scal_ref[:, 1:2]

    return pl.pallas_call(
        body, out_shape=jax.ShapeDtypeStruct((n_pad, DP), jnp.float32))


@functools.cache
def _final_matmul(n_pad, d_out):
    def body(parts_ref, scal_ref, w2_ref, b2_ref, out_ref):
        r = parts_ref[0] + parts_ref[1]
        out_ref[...] = (
            jnp.dot(r, w2_ref[...], preferred_element_type=jnp.float32)
            + scal_ref[:, 2:3] * b2_ref[...][None, :])

    return pl.pallas_call(
        body, out_shape=jax.ShapeDtypeStruct((n_pad, d_out), jnp.float32))


def kernel(feat, edge_index, eweight, W1, b1, W2, b2):
    n, d_in = feat.shape
    e = edge_index.shape[1]
    d_hid = W1.shape[1]
    d_out = W2.shape[1]

    n_pad = ((n + 1 + 127) // 128) * 128
    nch = -(-e // (NW * CHUNK))
    nch = ((nch + 3) // 4) * 4
    pt = nch * CHUNK
    e_pad = NW * pt

    src = edge_index[0]
    dst = edge_index[1]
    ew = eweight[:, 0]
    pad = e_pad - e
    srcp = jnp.concatenate([src, jnp.full((pad,), n, jnp.int32)])
    dstp = jnp.concatenate([dst, jnp.full((pad,), n, jnp.int32)])
    ewp = jnp.concatenate([ew, jnp.zeros((pad,), jnp.float32)])
    src2 = srcp.reshape(NW, pt)
    dst2 = dstp.reshape(NW, pt)
    ew2 = ewp.reshape(NW, pt)
    src3 = srcp.reshape(NW, nch, CHUNK)
    dst3 = dstp.reshape(NW, nch, CHUNK)
    ew3 = ewp.reshape(NW, nch, CHUNK)

    w1p = jnp.zeros((d_in, DP), jnp.float32).at[:, :d_hid].set(W1)
    b1p = jnp.zeros((DP,), jnp.float32).at[:d_hid].set(b1)
    w2p = jnp.zeros((DP, d_out), jnp.float32).at[:d_hid].set(W2)
    zz = jnp.zeros((n_pad, DP), jnp.float32)

    dp = _deg_kernel(n_pad, pt)(src2, dst2, ew2)
    x1, scal = _combine_matmul1(n_pad, n, d_in)(dp, feat, w1p)
    parts1 = _prop_kernel(n_pad, nch, False)(x1, src3, dst3, ew3, zz)
    x2 = _mid_elementwise(n_pad, n)(parts1, scal, b1p)
    parts2 = _prop_kernel(n_pad, nch, False)(x2, src3, dst3, ew3, zz)
    t = _pre_final_scale(n_pad)(parts2, scal)
    parts3 = _prop_kernel(n_pad, nch, True)(t, src3, dst3, ew3, zz)
    out = _final_matmul(n_pad, d_out)(parts3, scal, w2p, b2)
    return out[:n]


# speedup vs baseline: 1.0999x; 1.0999x over previous
import functools

import jax
import jax.numpy as jnp
from jax import lax
from jax.experimental import pallas as pl
from jax.experimental.pallas import tpu as pltpu
from jax.experimental.pallas import tpu_sc as plsc

NC = 2
NS = 16
L = 16
NW = NC * NS
CHUNK = 128
DP = 24
DH = 20


def _mesh():
    return plsc.VectorSubcoreMesh(
        core_axis_name="c", subcore_axis_name="s", num_cores=NC, num_subcores=NS
    )


@functools.cache
def _deg_kernel(n_pad, pt):
    def body(src_hbm, dst_hbm, ew_hbm, dp_hbm, src_v, dst_v, ew_v, acc):
        cid = lax.axis_index("c")
        sid = lax.axis_index("s")
        wid = sid * NC + cid
        z16 = jnp.zeros((L,), jnp.float32)

        def zero_body(i, carry):
            acc[pl.ds(i * L, L)] = z16
            return carry

        lax.fori_loop(0, (3 * n_pad) // L, zero_body, 0)

        pltpu.sync_copy(src_hbm.at[wid], src_v)
        pltpu.sync_copy(dst_hbm.at[wid], dst_v)
        pltpu.sync_copy(ew_hbm.at[wid], ew_v)

        ones = jnp.ones((L,), jnp.float32)
        off1 = jnp.int32(n_pad)
        off2 = jnp.int32(2 * n_pad)

        def hist_body(i, carry):
            s16 = src_v[pl.ds(i * L, L)]
            d16 = dst_v[pl.ds(i * L, L)]
            w16 = ew_v[pl.ds(i * L, L)]
            plsc.addupdate_scatter(acc, [s16], ones)
            plsc.addupdate_scatter(acc, [d16 + off1], ones)
            plsc.addupdate_scatter(acc, [d16 + off2], w16)
            return carry

        lax.fori_loop(0, pt // L, hist_body, 0)
        pltpu.sync_copy(acc, dp_hbm.at[wid])

    return pl.kernel(
        body,
        out_type=jax.ShapeDtypeStruct((NW, 3 * n_pad), jnp.float32),
        mesh=_mesh(),
        compiler_params=pltpu.CompilerParams(needs_layout_passes=False, use_tc_tiling_on_sc=False),
        scratch_types=[
            pltpu.VMEM((pt,), jnp.int32),
            pltpu.VMEM((pt,), jnp.int32),
            pltpu.VMEM((pt,), jnp.float32),
            pltpu.VMEM((3 * n_pad,), jnp.float32),
        ],
    )


@functools.cache
def _prop_kernel(n_pad, nch, with_ew):
    rpt = n_pad // NS
    assert nch % 4 == 0 and nch >= 8

    def body(x_hbm, src_hbm, dst_hbm, ew_hbm, zz_hbm, parts_hbm,
             sidx, didx, ewv, *scr):
        rows = scr[0:4]
        rows_m = scr[4:8] if with_ew else scr[0:4]
        acc = scr[-9]
        sg = scr[-8:-4]
        ss = scr[-4:]
        cid = lax.axis_index("c")
        sid = lax.axis_index("s")
        wid = sid * NC + cid

        pltpu.sync_copy(zz_hbm.at[pl.ds(sid * rpt, rpt)],
                        acc.at[pl.ds(sid * rpt, rpt)])
        pltpu.sync_copy(src_hbm.at[wid], sidx)
        pltpu.sync_copy(dst_hbm.at[wid], didx)
        if with_ew:
            pltpu.sync_copy(ew_hbm.at[wid], ewv)
        plsc.subcore_barrier()

        riota = lax.iota(jnp.int32, L)

        def gather(j, b):
            pltpu.async_copy(x_hbm.at[sidx.at[j]], rows[b], sg[b])

        def wait_gather(b):
            pltpu.make_async_copy(x_hbm.at[sidx.at[0]], rows[b], sg[b]).wait()

        def scatter(j, b):
            pltpu.async_copy(rows_m[b], acc.at[didx.at[j]], ss[b], add=True)

        def wait_scatter(b):
            pltpu.make_async_copy(rows_m[b], acc.at[didx.at[0]], ss[b]).wait()

        def mul_ew(j, b):
            for g in range(CHUNK // L):
                ew16 = ewv[j, pl.ds(g * L, L)]
                r16 = riota + jnp.int32(g * L)
                for c in range(DH):
                    c16 = jnp.full((L,), c, jnp.int32)
                    v = plsc.load_gather(rows[b], [r16, c16])
                    plsc.store_scatter(rows_m[b], [r16, c16], v * ew16)

        gather(0, 0)
        gather(1, 1)

        def outer(j0, carry):
            for db in range(4):
                j = j0 * 4 + db
                b = db
                wait_gather(b)
                if with_ew:
                    mul_ew(j, b)
                scatter(j, b)
                bn = (db + 2) % 4
                jn = j + 2

                def issue_next():
                    gather(jn, bn)

                if db >= 2:
                    @pl.when(jn < nch)
                    def _():
                        wait_scatter(bn)
                        issue_next()
                else:
                    @pl.when(jn < nch)
                    def _():
                        @pl.when(j0 >= 1)
                        def _():
                            wait_scatter(bn)
                        issue_next()
            return carry

        lax.fori_loop(0, nch // 4, outer, 0)
        for b in range(4):
            wait_scatter(b)
        plsc.subcore_barrier()
        pltpu.sync_copy(acc.at[pl.ds(sid * rpt, rpt)],
                        parts_hbm.at[cid, pl.ds(sid * rpt, rpt)])

    return pl.kernel(
        body,
        out_type=jax.ShapeDtypeStruct((NC, n_pad, DP), jnp.float32),
        mesh=_mesh(),
        compiler_params=pltpu.CompilerParams(needs_layout_passes=False, use_tc_tiling_on_sc=False),
        scratch_types=[
            pltpu.VMEM((nch, CHUNK), jnp.int32),
            pltpu.VMEM((nch, CHUNK), jnp.int32),
            pltpu.VMEM((nch, CHUNK), jnp.float32),
            *([pltpu.VMEM((CHUNK, DP), jnp.float32)] * (8 if with_ew else 4)),
            pltpu.VMEM_SHARED((n_pad, DP), jnp.float32),
            pltpu.SemaphoreType.DMA,
            pltpu.SemaphoreType.DMA,
            pltpu.SemaphoreType.DMA,
            pltpu.SemaphoreType.DMA,
            pltpu.SemaphoreType.DMA,
            pltpu.SemaphoreType.DMA,
            pltpu.SemaphoreType.DMA,
            pltpu.SemaphoreType.DMA,
        ],
    )


@functools.cache
def _combine_matmul1(n_pad, n, d_in):
    def body(dp_ref, feat_ref, w1_ref, x1_ref, scal_ref):
        d = jnp.sum(dp_ref[...], axis=0)
        s_out = lax.rsqrt(jnp.maximum(d[0], 1.0))
        s_in = lax.rsqrt(jnp.maximum(d[1], 1.0))
        x1_ref[...] = jnp.dot(feat_ref[...] * s_out[:, None], w1_ref[...],
                              preferred_element_type=jnp.float32)
        scal_ref[...] = jnp.stack(
            [s_out, s_in, d[2], jnp.zeros_like(s_out)], axis=1)

    return pl.pallas_call(
        body,
        out_shape=(
            jax.ShapeDtypeStruct((n_pad, DP), jnp.float32),
            jax.ShapeDtypeStruct((n_pad, 4), jnp.float32),
        ),
    )


@functools.cache
def _mid_elementwise(n_pad, n):
    def body(parts_ref, scal_ref, b1_ref, x2_ref):
        p = parts_ref[0] + parts_ref[1]
        h = jnp.maximum(p * scal_ref[:, 1:2] + b1_ref[...][None, :], 0.0)
        h = h * scal_ref[:, 0:1]
        rmask = lax.broadcasted_iota(jnp.int32, (n_pad, 1), 0) < n
        x2_ref[...] = jnp.where(rmask, h, 0.0)

    return pl.pallas_call(
        body, out_shape=jax.ShapeDtypeStruct((n_pad, DP), jnp.float32))


@functools.cache
def _pre_final_scale(n_pad):
    def body(parts_ref, scal_ref, t_ref):
        t_ref[...] = (parts_ref[0] + parts_ref[1]) * scal_ref[:, 1:2]

    return pl.pallas_call(
        body, out_shape=jax.ShapeDtypeStruct((n_pad, DP), jnp.float32))


@functools.cache
def _final_matmul(n_pad, d_out):
    def body(parts_ref, scal_ref, w2_ref, b2_ref, out_ref):
        r = parts_ref[0] + parts_ref[1]
        out_ref[...] = (
            jnp.dot(r, w2_ref[...], preferred_element_type=jnp.float32)
            + scal_ref[:, 2:3] * b2_ref[...][None, :])

    return pl.pallas_call(
        body, out_shape=jax.ShapeDtypeStruct((n_pad, d_out), jnp.float32))


def kernel(feat, edge_index, eweight, W1, b1, W2, b2):
    n, d_in = feat.shape
    e = edge_index.shape[1]
    d_hid = W1.shape[1]
    d_out = W2.shape[1]

    n_pad = ((n + 1 + 127) // 128) * 128
    nch = -(-e // (NW * CHUNK))
    nch = ((nch + 3) // 4) * 4
    pt = nch * CHUNK
    e_pad = NW * pt

    src = edge_index[0]
    dst = edge_index[1]
    ew = eweight[:, 0]
    pad = e_pad - e
    srcp = jnp.concatenate([src, jnp.full((pad,), n, jnp.int32)])
    dstp = jnp.concatenate([dst, jnp.full((pad,), n, jnp.int32)])
    ewp = jnp.concatenate([ew, jnp.zeros((pad,), jnp.float32)])
    src2 = srcp.reshape(NW, pt)
    dst2 = dstp.reshape(NW, pt)
    ew2 = ewp.reshape(NW, pt)
    src3 = srcp.reshape(NW, nch, CHUNK)
    dst3 = dstp.reshape(NW, nch, CHUNK)
    ew3 = ewp.reshape(NW, nch, CHUNK)

    feat_p = jnp.zeros((n_pad, d_in), jnp.float32).at[:n].set(feat)
    w1p = jnp.zeros((d_in, DP), jnp.float32).at[:, :d_hid].set(W1)
    b1p = jnp.zeros((DP,), jnp.float32).at[:d_hid].set(b1)
    w2p = jnp.zeros((DP, d_out), jnp.float32).at[:d_hid].set(W2)
    zz = jnp.zeros((n_pad, DP), jnp.float32)

    dp = _deg_kernel(n_pad, pt)(src2, dst2, ew2)
    x1, scal = _combine_matmul1(n_pad, n, d_in)(
        dp.reshape(NW, 3, n_pad), feat_p, w1p)
    parts1 = _prop_kernel(n_pad, nch, False)(x1, src3, dst3, ew3, zz)
    x2 = _mid_elementwise(n_pad, n)(parts1, scal, b1p)
    parts2 = _prop_kernel(n_pad, nch, False)(x2, src3, dst3, ew3, zz)
    t = _pre_final_scale(n_pad)(parts2, scal)
    parts3 = _prop_kernel(n_pad, nch, True)(t, src3, dst3, ew3, zz)
    out = _final_matmul(n_pad, d_out)(parts3, scal, w2p, b2)
    return out[:n]


# speedup vs baseline: 1.1612x; 1.0557x over previous
import functools

import jax
import jax.numpy as jnp
from jax import lax
from jax.experimental import pallas as pl
from jax.experimental.pallas import tpu as pltpu
from jax.experimental.pallas import tpu_sc as plsc

NC = 2
NS = 16
L = 16
NW = NC * NS
CHUNK = 128
DP = 24
DH = 20


def _mesh():
    return plsc.VectorSubcoreMesh(
        core_axis_name="c", subcore_axis_name="s", num_cores=NC, num_subcores=NS
    )


@functools.cache
def _deg_kernel(n_pad, pt):
    def body(src_hbm, dst_hbm, ew_hbm, dp_hbm, src_v, dst_v, ew_v, acc):
        cid = lax.axis_index("c")
        sid = lax.axis_index("s")
        wid = sid * NC + cid
        z16 = jnp.zeros((L,), jnp.float32)

        def zero_body(i, carry):
            acc[pl.ds(i * L, L)] = z16
            return carry

        lax.fori_loop(0, (3 * n_pad) // L, zero_body, 0)

        pltpu.sync_copy(src_hbm.at[wid], src_v)
        pltpu.sync_copy(dst_hbm.at[wid], dst_v)
        pltpu.sync_copy(ew_hbm.at[wid], ew_v)

        ones = jnp.ones((L,), jnp.float32)
        off1 = jnp.int32(n_pad)
        off2 = jnp.int32(2 * n_pad)

        def hist_body(i, carry):
            s16 = src_v[pl.ds(i * L, L)]
            d16 = dst_v[pl.ds(i * L, L)]
            w16 = ew_v[pl.ds(i * L, L)]
            plsc.addupdate_scatter(acc, [s16], ones)
            plsc.addupdate_scatter(acc, [d16 + off1], ones)
            plsc.addupdate_scatter(acc, [d16 + off2], w16)
            return carry

        lax.fori_loop(0, pt // L, hist_body, 0)
        pltpu.sync_copy(acc, dp_hbm.at[wid])

    return pl.kernel(
        body,
        out_type=jax.ShapeDtypeStruct((NW, 3 * n_pad), jnp.float32),
        mesh=_mesh(),
        compiler_params=pltpu.CompilerParams(needs_layout_passes=False, use_tc_tiling_on_sc=False),
        scratch_types=[
            pltpu.VMEM((pt,), jnp.int32),
            pltpu.VMEM((pt,), jnp.int32),
            pltpu.VMEM((pt,), jnp.float32),
            pltpu.VMEM((3 * n_pad,), jnp.float32),
        ],
    )


@functools.cache
def _prop_kernel(n_pad, nch, with_ew):
    rpt = n_pad // NS
    assert nch % 4 == 0 and nch >= 8

    def body(x_hbm, src_hbm, dst_hbm, ew_hbm, zz_hbm, parts_hbm,
             sidx, didx, ewv, *scr):
        rows = scr[0:4]
        rows_m = scr[4:8] if with_ew else scr[0:4]
        acc = scr[-9]
        sg = scr[-8:-4]
        ss = scr[-4:]
        cid = lax.axis_index("c")
        sid = lax.axis_index("s")
        wid = sid * NC + cid

        pltpu.sync_copy(zz_hbm.at[pl.ds(sid * rpt, rpt)],
                        acc.at[pl.ds(sid * rpt, rpt)])
        pltpu.sync_copy(src_hbm.at[wid], sidx)
        pltpu.sync_copy(dst_hbm.at[wid], didx)
        if with_ew:
            pltpu.sync_copy(ew_hbm.at[wid], ewv)
        plsc.subcore_barrier()

        riota = lax.iota(jnp.int32, L)

        def gather(j, b):
            pltpu.async_copy(x_hbm.at[sidx.at[j]], rows[b], sg[b])

        def wait_gather(b):
            pltpu.make_async_copy(x_hbm.at[sidx.at[0]], rows[b], sg[b]).wait()

        def scatter(j, b):
            pltpu.async_copy(rows_m[b], acc.at[didx.at[j]], ss[b], add=True)

        def wait_scatter(b):
            pltpu.make_async_copy(rows_m[b], acc.at[didx.at[0]], ss[b]).wait()

        def mul_ew(j, b):
            for g in range(CHUNK // L):
                ew16 = ewv[j, pl.ds(g * L, L)]
                r16 = riota + jnp.int32(g * L)
                for c in range(DH):
                    c16 = jnp.full((L,), c, jnp.int32)
                    v = plsc.load_gather(rows[b], [r16, c16])
                    plsc.store_scatter(rows_m[b], [r16, c16], v * ew16)

        gather(0, 0)
        gather(1, 1)

        def outer(j0, carry):
            for db in range(4):
                j = j0 * 4 + db
                b = db
                wait_gather(b)
                if with_ew:
                    mul_ew(j, b)
                scatter(j, b)
                bn = (db + 2) % 4
                jn = j + 2

                def issue_next():
                    gather(jn, bn)

                if db >= 2:
                    @pl.when(jn < nch)
                    def _():
                        wait_scatter(bn)
                        issue_next()
                else:
                    @pl.when(jn < nch)
                    def _():
                        @pl.when(j0 >= 1)
                        def _():
                            wait_scatter(bn)
                        issue_next()
            return carry

        lax.fori_loop(0, nch // 4, outer, 0)
        for b in range(4):
            wait_scatter(b)
        plsc.subcore_barrier()
        pltpu.sync_copy(acc.at[pl.ds(sid * rpt, rpt)],
                        parts_hbm.at[cid, pl.ds(sid * rpt, rpt)])

    return pl.kernel(
        body,
        out_type=jax.ShapeDtypeStruct((NC, n_pad, DP), jnp.float32),
        mesh=_mesh(),
        compiler_params=pltpu.CompilerParams(needs_layout_passes=False, use_tc_tiling_on_sc=False),
        scratch_types=[
            pltpu.VMEM((nch, CHUNK), jnp.int32),
            pltpu.VMEM((nch, CHUNK), jnp.int32),
            pltpu.VMEM((nch, CHUNK), jnp.float32),
            *([pltpu.VMEM((CHUNK, DP), jnp.float32)] * (8 if with_ew else 4)),
            pltpu.VMEM_SHARED((n_pad, DP), jnp.float32),
            pltpu.SemaphoreType.DMA,
            pltpu.SemaphoreType.DMA,
            pltpu.SemaphoreType.DMA,
            pltpu.SemaphoreType.DMA,
            pltpu.SemaphoreType.DMA,
            pltpu.SemaphoreType.DMA,
            pltpu.SemaphoreType.DMA,
            pltpu.SemaphoreType.DMA,
        ],
    )


@functools.cache
def _combine_matmul1(n_pad, n, d_in):
    def body(dp_ref, feat_ref, w1_ref, x1_ref, scal_ref):
        d = jnp.sum(dp_ref[...], axis=0)
        s_out = lax.rsqrt(jnp.maximum(d[0], 1.0))
        s_in = lax.rsqrt(jnp.maximum(d[1], 1.0))
        x1_ref[...] = jnp.dot(feat_ref[...] * s_out[:, None], w1_ref[...],
                              preferred_element_type=jnp.float32)
        scal_ref[...] = jnp.stack(
            [s_out, s_in, d[2], jnp.zeros_like(s_out)], axis=1)

    return pl.pallas_call(
        body,
        out_shape=(
            jax.ShapeDtypeStruct((n_pad, DP), jnp.float32),
            jax.ShapeDtypeStruct((n_pad, 4), jnp.float32),
        ),
    )


@functools.cache
def _mid_elementwise(n_pad, n):
    def body(parts_ref, scal_ref, b1_ref, x2_ref):
        p = parts_ref[0] + parts_ref[1]
        h = jnp.maximum(p * scal_ref[:, 1:2] + b1_ref[...][None, :], 0.0)
        h = h * scal_ref[:, 0:1]
        rmask = lax.broadcasted_iota(jnp.int32, (n_pad, 1), 0) < n
        x2_ref[...] = jnp.where(rmask, h, 0.0)

    return pl.pallas_call(
        body, out_shape=jax.ShapeDtypeStruct((n_pad, DP), jnp.float32))


@functools.cache
def _pre_final_scale(n_pad):
    def body(parts_ref, scal_ref, t_ref):
        t_ref[...] = (parts_ref[0] + parts_ref[1]) * scal_ref[:, 1:2]

    return pl.pallas_call(
        body, out_shape=jax.ShapeDtypeStruct((n_pad, DP), jnp.float32))


@functools.cache
def _final_matmul(n_pad, n, d_out):
    def body(parts_ref, scal_ref, w2_ref, b2_ref, out_ref):
        r = parts_ref[0, pl.ds(0, n)] + parts_ref[1, pl.ds(0, n)]
        out_ref[...] = (
            jnp.dot(r, w2_ref[...], preferred_element_type=jnp.float32)
            + scal_ref[pl.ds(0, n), 2:3] * b2_ref[...][None, :])

    return pl.pallas_call(
        body, out_shape=jax.ShapeDtypeStruct((n, d_out), jnp.float32))


def kernel(feat, edge_index, eweight, W1, b1, W2, b2):
    n, d_in = feat.shape
    e = edge_index.shape[1]
    d_hid = W1.shape[1]
    d_out = W2.shape[1]

    n_pad = ((n + 1 + 127) // 128) * 128
    nch = -(-e // (NW * CHUNK))
    nch = ((nch + 3) // 4) * 4
    pt = nch * CHUNK
    e_pad = NW * pt

    src = edge_index[0]
    dst = edge_index[1]
    ew = eweight[:, 0]
    pad = e_pad - e
    srcp = jnp.concatenate([src, jnp.full((pad,), n, jnp.int32)])
    dstp = jnp.concatenate([dst, jnp.full((pad,), n, jnp.int32)])
    ewp = jnp.concatenate([ew, jnp.zeros((pad,), jnp.float32)])
    src2 = srcp.reshape(NW, pt)
    dst2 = dstp.reshape(NW, pt)
    ew2 = ewp.reshape(NW, pt)
    src3 = srcp.reshape(NW, nch, CHUNK)
    dst3 = dstp.reshape(NW, nch, CHUNK)
    ew3 = ewp.reshape(NW, nch, CHUNK)

    feat_p = jnp.zeros((n_pad, d_in), jnp.float32).at[:n].set(feat)
    w1p = jnp.zeros((d_in, DP), jnp.float32).at[:, :d_hid].set(W1)
    b1p = jnp.zeros((DP,), jnp.float32).at[:d_hid].set(b1)
    w2p = jnp.zeros((DP, d_out), jnp.float32).at[:d_hid].set(W2)
    zz = jnp.zeros((n_pad, DP), jnp.float32)

    dp = _deg_kernel(n_pad, pt)(src2, dst2, ew2)
    x1, scal = _combine_matmul1(n_pad, n, d_in)(
        dp.reshape(NW, 3, n_pad), feat_p, w1p)
    parts1 = _prop_kernel(n_pad, nch, False)(x1, src3, dst3, ew3, zz)
    rmask = (jnp.arange(n_pad) < n)[:, None]
    x2 = jnp.where(
        rmask,
        jnp.maximum((parts1[0] + parts1[1]) * scal[:, 1:2] + b1p[None, :], 0.0)
        * scal[:, 0:1],
        0.0)
    parts2 = _prop_kernel(n_pad, nch, False)(x2, src3, dst3, ew3, zz)
    t = (parts2[0] + parts2[1]) * scal[:, 1:2]
    parts3 = _prop_kernel(n_pad, nch, True)(t, src3, dst3, ew3, zz)
    return _final_matmul(n_pad, n, d_out)(parts3, scal, w2p, b2)
